# Initial kernel scaffold; baseline (speedup 1.0000x reference)
import jax
import jax.numpy as jnp
from jax.experimental import pallas as pl


def kernel(x, W1, b1, W2, b2):
    raise NotImplementedError("write your pallas kernel here")


# speedup vs baseline: 3.0691x; 3.0691x over previous
import functools

import jax
import jax.numpy as jnp
from jax.experimental import pallas as pl

_BLOCK = 1024


def _router_block(x_ref, w1_ref, b1_ref, w2_ref, b2_ref, alpha_ref, logits_ref):
    h = jnp.dot(x_ref[...], w1_ref[...], preferred_element_type=jnp.float32)
    h = h + b1_ref[...]
    h = jnp.where(h > 0, h, jnp.exp(jnp.minimum(h, 0.0)) - 1.0)
    logits = jnp.dot(h, w2_ref[...], preferred_element_type=jnp.float32)
    logits = logits + b2_ref[...]

    n, e = logits.shape
    j = jax.lax.broadcasted_iota(jnp.int32, (n, e), 1)
    neg_inf = jnp.float32(-jnp.inf)

    m1 = jnp.max(logits, axis=1, keepdims=True)
    idx1 = jnp.min(jnp.where(logits == m1, j, e), axis=1, keepdims=True)
    keep1 = j == idx1

    rest = jnp.where(keep1, neg_inf, logits)
    m2 = jnp.max(rest, axis=1, keepdims=True)
    idx2 = jnp.min(jnp.where(rest == m2, j, e), axis=1, keepdims=True)
    keep = keep1 | (j == idx2)

    e_val = jnp.where(keep, jnp.exp(logits - m1), 0.0)
    alpha = e_val / jnp.sum(e_val, axis=1, keepdims=True)

    logits_ref[...] = logits
    alpha_ref[...] = alpha


@jax.jit
def kernel(x, W1, b1, W2, b2):
    n_tokens, in_dim = x.shape
    hidden = W1.shape[1]
    n_exp = W2.shape[1]
    grid = (n_tokens // _BLOCK,)
    alpha, logits = pl.pallas_call(
        _router_block,
        grid=grid,
        in_specs=[
            pl.BlockSpec((_BLOCK, in_dim), lambda i: (i, 0)),
            pl.BlockSpec((in_dim, hidden), lambda i: (0, 0)),
            pl.BlockSpec((hidden,), lambda i: (0,)),
            pl.BlockSpec((hidden, n_exp), lambda i: (0, 0)),
            pl.BlockSpec((n_exp,), lambda i: (0,)),
        ],
        out_specs=[
            pl.BlockSpec((_BLOCK, n_exp), lambda i: (i, 0)),
            pl.BlockSpec((_BLOCK, n_exp), lambda i: (i, 0)),
        ],
        out_shape=[
            jax.ShapeDtypeStruct((n_tokens, n_exp), jnp.float32),
            jax.ShapeDtypeStruct((n_tokens, n_exp), jnp.float32),
        ],
    )(x, W1, b1, W2, b2)
    return alpha, logits


# speedup vs baseline: 3.2260x; 1.0511x over previous
import functools

import jax
import jax.numpy as jnp
from jax.experimental import pallas as pl

_BLOCK = 2048


def _router_block(x_ref, w1_ref, b1_ref, w2_ref, b2_ref, alpha_ref, logits_ref):
    h = jnp.dot(x_ref[...], w1_ref[...], preferred_element_type=jnp.float32)
    h = h + b1_ref[...]
    h = jnp.where(h > 0, h, jnp.exp(jnp.minimum(h, 0.0)) - 1.0)
    logits = jnp.dot(h, w2_ref[...], preferred_element_type=jnp.float32)
    logits = logits + b2_ref[...]

    n, e = logits.shape
    j = jax.lax.broadcasted_iota(jnp.int32, (n, e), 1)
    neg_inf = jnp.float32(-jnp.inf)

    m1 = jnp.max(logits, axis=1, keepdims=True)
    idx1 = jnp.min(jnp.where(logits == m1, j, e), axis=1, keepdims=True)
    keep1 = j == idx1

    rest = jnp.where(keep1, neg_inf, logits)
    m2 = jnp.max(rest, axis=1, keepdims=True)
    idx2 = jnp.min(jnp.where(rest == m2, j, e), axis=1, keepdims=True)
    keep = keep1 | (j == idx2)

    e_val = jnp.where(keep, jnp.exp(logits - m1), 0.0)
    alpha = e_val / jnp.sum(e_val, axis=1, keepdims=True)

    logits_ref[...] = logits
    alpha_ref[...] = alpha


@jax.jit
def kernel(x, W1, b1, W2, b2):
    n_tokens, in_dim = x.shape
    hidden = W1.shape[1]
    n_exp = W2.shape[1]
    grid = (n_tokens // _BLOCK,)
    alpha, logits = pl.pallas_call(
        _router_block,
        grid=grid,
        in_specs=[
            pl.BlockSpec((_BLOCK, in_dim), lambda i: (i, 0)),
            pl.BlockSpec((in_dim, hidden), lambda i: (0, 0)),
            pl.BlockSpec((hidden,), lambda i: (0,)),
            pl.BlockSpec((hidden, n_exp), lambda i: (0, 0)),
            pl.BlockSpec((n_exp,), lambda i: (0,)),
        ],
        out_specs=[
            pl.BlockSpec((_BLOCK, n_exp), lambda i: (i, 0)),
            pl.BlockSpec((_BLOCK, n_exp), lambda i: (i, 0)),
        ],
        out_shape=[
            jax.ShapeDtypeStruct((n_tokens, n_exp), jnp.float32),
            jax.ShapeDtypeStruct((n_tokens, n_exp), jnp.float32),
        ],
    )(x, W1, b1, W2, b2)
    return alpha, logits


# speedup vs baseline: 3.2260x; 1.0000x over previous
import functools

import jax
import jax.numpy as jnp
from jax.experimental import pallas as pl
from jax.experimental.pallas import tpu as pltpu

_BLOCK = 2048


def _router_block(x_ref, w1_ref, b1_ref, w2_ref, b2_ref, alpha_ref, logits_ref):
    h = jnp.dot(x_ref[...], w1_ref[...], preferred_element_type=jnp.float32)
    h = h + b1_ref[...]
    h = jnp.where(h > 0, h, jnp.exp(jnp.minimum(h, 0.0)) - 1.0)
    logits = jnp.dot(h, w2_ref[...], preferred_element_type=jnp.float32)
    logits = logits + b2_ref[...]

    n, e = logits.shape
    j = jax.lax.broadcasted_iota(jnp.int32, (n, e), 1)
    neg_inf = jnp.float32(-jnp.inf)

    m1 = jnp.max(logits, axis=1, keepdims=True)
    idx1 = jnp.min(jnp.where(logits == m1, j, e), axis=1, keepdims=True)
    keep1 = j == idx1

    rest = jnp.where(keep1, neg_inf, logits)
    m2 = jnp.max(rest, axis=1, keepdims=True)
    idx2 = jnp.min(jnp.where(rest == m2, j, e), axis=1, keepdims=True)
    keep = keep1 | (j == idx2)

    e_val = jnp.where(keep, jnp.exp(logits - m1), 0.0)
    alpha = e_val / jnp.sum(e_val, axis=1, keepdims=True)

    logits_ref[...] = logits
    alpha_ref[...] = alpha


@jax.jit
def kernel(x, W1, b1, W2, b2):
    n_tokens, in_dim = x.shape
    hidden = W1.shape[1]
    n_exp = W2.shape[1]
    grid = (n_tokens // _BLOCK,)
    alpha, logits = pl.pallas_call(
        _router_block,
        grid=grid,
        in_specs=[
            pl.BlockSpec((_BLOCK, in_dim), lambda i: (i, 0)),
            pl.BlockSpec((in_dim, hidden), lambda i: (0, 0)),
            pl.BlockSpec((hidden,), lambda i: (0,)),
            pl.BlockSpec((hidden, n_exp), lambda i: (0, 0)),
            pl.BlockSpec((n_exp,), lambda i: (0,)),
        ],
        out_specs=[
            pl.BlockSpec((_BLOCK, n_exp), lambda i: (i, 0)),
            pl.BlockSpec((_BLOCK, n_exp), lambda i: (i, 0)),
        ],
        out_shape=[
            jax.ShapeDtypeStruct((n_tokens, n_exp), jnp.float32),
            jax.ShapeDtypeStruct((n_tokens, n_exp), jnp.float32),
        ],
        compiler_params=pltpu.CompilerParams(
            dimension_semantics=("parallel",),
        ),
    )(x, W1, b1, W2, b2)
    return alpha, logits
